# baseline (speedup 1.0000x reference)
import functools

import jax
import jax.numpy as jnp
from jax import lax
from jax.experimental import pallas as pl
from jax.experimental.pallas import tpu as pltpu
from jax.experimental.pallas import tpu_sc as plsc

_NC = 2
_NS = 16
_NW = _NC * _NS

_VB = 512
_NEG = -1e30


def _embed_bag(B, CTX, V, E):
  rows_per_w = B // _NW
  idx_per_w = rows_per_w * CTX
  chunk = 64
  n_chunks = idx_per_w // chunk

  mesh = plsc.VectorSubcoreMesh(core_axis_name="c", subcore_axis_name="s")

  @functools.partial(
      pl.kernel,
      out_type=[
          jax.ShapeDtypeStruct((B, E), jnp.float32),
          jax.ShapeDtypeStruct((B, E), jnp.float32),
      ],
      mesh=mesh,
      scratch_types=[
          pltpu.VMEM((idx_per_w,), jnp.int32),
          pltpu.VMEM((idx_per_w, E), jnp.float32),
          pltpu.VMEM((rows_per_w,), jnp.int32),
          pltpu.VMEM((rows_per_w, E), jnp.float32),
          pltpu.VMEM((rows_per_w, E), jnp.float32),
          pltpu.SemaphoreType.DMA,
      ],
      compiler_params=pltpu.CompilerParams(use_tc_tiling_on_sc=False),
  )
  def body(x_hbm, y_hbm, table_hbm, w_hbm, sumx_hbm, wy_hbm,
           idx_v, rows_v, yidx_v, wrows_v, acc_v, sem):
    wid = lax.axis_index("s") * _NC + lax.axis_index("c")
    xbase = wid * idx_per_w
    rbase = wid * rows_per_w

    pltpu.sync_copy(x_hbm.at[pl.ds(xbase, idx_per_w)], idx_v)
    pltpu.sync_copy(y_hbm.at[pl.ds(rbase, rows_per_w)], yidx_v)

    copies = []
    for c in range(n_chunks):
      copies.append(pltpu.async_copy(
          table_hbm.at[idx_v.at[pl.ds(c * chunk, chunk)]],
          rows_v.at[pl.ds(c * chunk, chunk)], sem))
    copies.append(pltpu.async_copy(w_hbm.at[yidx_v], wrows_v, sem))
    for cp in copies:
      cp.wait()

    def accum_row(r, _):
      for c in range(E // 16):
        sl = pl.ds(c * 16, 16)
        a = rows_v[r * CTX, sl]
        for j in range(1, CTX):
          a = a + rows_v[r * CTX + j, sl]
        acc_v[r, sl] = a
      return _

    lax.fori_loop(0, rows_per_w, accum_row, 0)

    pltpu.sync_copy(acc_v, sumx_hbm.at[pl.ds(rbase, rows_per_w)])
    pltpu.sync_copy(wrows_v, wy_hbm.at[pl.ds(rbase, rows_per_w)])

  return body


def _fused_loss(B, V, E, interpret=False):
  nv = pl.cdiv(V, _VB)

  def body(sumx_ref, w_ref, wy_ref, loss_ref, m_ref, s_ref):
    v = pl.program_id(0)

    @pl.when(v == 0)
    def _init():
      m_ref[...] = jnp.full(m_ref.shape, _NEG, jnp.float32)
      s_ref[...] = jnp.zeros(s_ref.shape, jnp.float32)

    z = lax.dot_general(sumx_ref[...], w_ref[...], (((1,), (1,)), ((), ())),
                        preferred_element_type=jnp.float32)
    col = v * _VB + lax.broadcasted_iota(jnp.int32, z.shape, 1)
    z = jnp.where(col < V, z, _NEG)
    m_old = m_ref[...]
    m_new = jnp.maximum(m_old, jnp.max(z, axis=1, keepdims=True))
    s_ref[...] = (s_ref[...] * jnp.exp(m_old - m_new)
                  + jnp.sum(jnp.exp(z - m_new), axis=1, keepdims=True))
    m_ref[...] = m_new

    @pl.when(v == nv - 1)
    def _fin():
      lse = m_ref[...] + jnp.log(s_ref[...])
      picked = jnp.sum(sumx_ref[...] * wy_ref[...], axis=1, keepdims=True)
      loss_ref[0, 0] = jnp.mean(lse - picked)

  return pl.pallas_call(
      body,
      grid=(nv,),
      in_specs=[
          pl.BlockSpec((B, E), lambda v: (0, 0)),
          pl.BlockSpec((_VB, E), lambda v: (v, 0)),
          pl.BlockSpec((B, E), lambda v: (0, 0)),
      ],
      out_specs=pl.BlockSpec((1, 1), lambda v: (0, 0),
                             memory_space=pltpu.SMEM),
      out_shape=jax.ShapeDtypeStruct((1, 1), jnp.float32),
      scratch_shapes=[pltpu.VMEM((B, 1), jnp.float32),
                      pltpu.VMEM((B, 1), jnp.float32)],
      interpret=interpret,
  )


def kernel(batch_X, batch_Y, emb_table, linear_W):
  B, CTX = batch_X.shape
  V, E = emb_table.shape
  x_flat = batch_X.reshape(-1).astype(jnp.int32)
  y = batch_Y.astype(jnp.int32)
  sum_x, w_y = _embed_bag(B, CTX, V, E)(x_flat, y, emb_table, linear_W)
  loss = _fused_loss(B, V, E)(sum_x, linear_W, w_y)
  return loss[0, 0]


# speedup vs baseline: 2.4439x; 2.4439x over previous
import functools

import jax
import jax.numpy as jnp
from jax import lax
from jax.experimental import pallas as pl
from jax.experimental.pallas import tpu as pltpu
from jax.experimental.pallas import tpu_sc as plsc

_NC = 2
_NS = 16
_NW = _NC * _NS

_VB = 1000


def _embed_bag(B, CTX, V, E):
  rows_per_w = B // _NW
  idx_per_w = rows_per_w * CTX
  chunk = 64
  n_chunks = idx_per_w // chunk

  mesh = plsc.VectorSubcoreMesh(core_axis_name="c", subcore_axis_name="s")

  @functools.partial(
      pl.kernel,
      out_type=[
          jax.ShapeDtypeStruct((B, E), jnp.float32),
          jax.ShapeDtypeStruct((B, E), jnp.float32),
      ],
      mesh=mesh,
      scratch_types=[
          pltpu.VMEM((idx_per_w,), jnp.int32),
          pltpu.VMEM((idx_per_w, E), jnp.float32),
          pltpu.VMEM((rows_per_w,), jnp.int32),
          pltpu.VMEM((rows_per_w, E), jnp.float32),
          pltpu.VMEM((rows_per_w, E), jnp.float32),
          pltpu.SemaphoreType.DMA,
      ],
      compiler_params=pltpu.CompilerParams(use_tc_tiling_on_sc=False),
  )
  def body(x_hbm, y_hbm, table_hbm, w_hbm, sumx_hbm, wy_hbm,
           idx_v, rows_v, yidx_v, wrows_v, acc_v, sem):
    wid = lax.axis_index("s") * _NC + lax.axis_index("c")
    xbase = wid * idx_per_w
    rbase = wid * rows_per_w

    pltpu.sync_copy(x_hbm.at[pl.ds(xbase, idx_per_w)], idx_v)
    pltpu.sync_copy(y_hbm.at[pl.ds(rbase, rows_per_w)], yidx_v)

    copies = []
    for c in range(n_chunks):
      copies.append(pltpu.async_copy(
          table_hbm.at[idx_v.at[pl.ds(c * chunk, chunk)]],
          rows_v.at[pl.ds(c * chunk, chunk)], sem))
    copies.append(pltpu.async_copy(w_hbm.at[yidx_v], wrows_v, sem))
    for cp in copies:
      cp.wait()

    def accum_row(r, _):
      for c in range(E // 16):
        sl = pl.ds(c * 16, 16)
        a = rows_v[r * CTX, sl]
        for j in range(1, CTX):
          a = a + rows_v[r * CTX + j, sl]
        acc_v[r, sl] = a
      return _

    lax.fori_loop(0, rows_per_w, accum_row, 0)

    pltpu.sync_copy(acc_v, sumx_hbm.at[pl.ds(rbase, rows_per_w)])
    pltpu.sync_copy(wrows_v, wy_hbm.at[pl.ds(rbase, rows_per_w)])

  return body


def _fused_loss(B, V, E, interpret=False):
  assert V % _VB == 0
  nv = V // _VB
  bound = 1.0 / (E ** 0.5)

  def body(sumx_ref, w_ref, wy_ref, loss_ref, rhs_ref, s_ref):
    v = pl.program_id(0)

    @pl.when(v == 0)
    def _init():
      sx = sumx_ref[...]
      r = lax.broadcasted_iota(jnp.int32, (E, E), 0)
      c = lax.broadcasted_iota(jnp.int32, (E, E), 1)
      eye = (r == c).astype(jnp.float32)
      sxT = lax.dot_general(eye, sx, (((1,), (1,)), ((), ())),
                            preferred_element_type=jnp.float32)
      ones_row = jnp.ones((1, E), jnp.float32)
      m = bound * lax.dot_general(ones_row, jnp.abs(sx),
                                  (((1,), (1,)), ((), ())),
                                  preferred_element_type=jnp.float32)
      rhs_ref[pl.ds(0, E), :] = sxT
      rhs_ref[pl.ds(E, 1), :] = -m
      s_ref[...] = jnp.zeros(s_ref.shape, jnp.float32)

    w_aug = jnp.concatenate(
        [w_ref[...], jnp.ones((_VB, 1), jnp.float32)], axis=1)
    zm = lax.dot_general(w_aug, rhs_ref[...], (((1,), (0,)), ((), ())),
                         preferred_element_type=jnp.float32)
    s_ref[...] = s_ref[...] + jnp.sum(jnp.exp(zm), axis=0, keepdims=True)

    @pl.when(v == nv - 1)
    def _fin():
      ones_row = jnp.ones((1, E), jnp.float32)
      pk = lax.dot_general(ones_row, sumx_ref[...] * wy_ref[...],
                           (((1,), (1,)), ((), ())),
                           preferred_element_type=jnp.float32)
      m = -rhs_ref[pl.ds(E, 1), :]
      lse = m + jnp.log(s_ref[...])
      loss_ref[0, 0] = jnp.mean(lse - pk)

  return pl.pallas_call(
      body,
      grid=(nv,),
      in_specs=[
          pl.BlockSpec((B, E), lambda v: (0, 0)),
          pl.BlockSpec((_VB, E), lambda v: (v, 0)),
          pl.BlockSpec((B, E), lambda v: (0, 0)),
      ],
      out_specs=pl.BlockSpec((1, 1), lambda v: (0, 0),
                             memory_space=pltpu.SMEM),
      out_shape=jax.ShapeDtypeStruct((1, 1), jnp.float32),
      scratch_shapes=[pltpu.VMEM((E + 1, B), jnp.float32),
                      pltpu.VMEM((1, B), jnp.float32)],
      interpret=interpret,
  )


def kernel(batch_X, batch_Y, emb_table, linear_W):
  B, CTX = batch_X.shape
  V, E = emb_table.shape
  x_flat = batch_X.reshape(-1).astype(jnp.int32)
  y = batch_Y.astype(jnp.int32)
  sum_x, w_y = _embed_bag(B, CTX, V, E)(x_flat, y, emb_table, linear_W)
  loss = _fused_loss(B, V, E)(sum_x, linear_W, w_y)
  return loss[0, 0]


# speedup vs baseline: 2.4532x; 1.0038x over previous
import functools

import jax
import jax.numpy as jnp
from jax import lax
from jax.experimental import pallas as pl
from jax.experimental.pallas import tpu as pltpu
from jax.experimental.pallas import tpu_sc as plsc

_NC = 2
_NS = 16
_NW = _NC * _NS

_VB = 1000


def _embed_bag(B, CTX, V, E):
  rows_per_w = B // _NW
  idx_per_w = rows_per_w * CTX
  chunk = 64
  n_chunks = idx_per_w // chunk

  mesh = plsc.VectorSubcoreMesh(core_axis_name="c", subcore_axis_name="s")

  @functools.partial(
      pl.kernel,
      out_type=[
          jax.ShapeDtypeStruct((B, E), jnp.float32),
          jax.ShapeDtypeStruct((B, E), jnp.float32),
      ],
      mesh=mesh,
      scratch_types=[
          pltpu.VMEM((idx_per_w,), jnp.int32),
          pltpu.VMEM((idx_per_w, E), jnp.float32),
          pltpu.VMEM((rows_per_w,), jnp.int32),
          pltpu.VMEM((rows_per_w, E), jnp.float32),
          pltpu.VMEM((rows_per_w, E), jnp.float32),
          pltpu.SemaphoreType.DMA,
      ],
      compiler_params=pltpu.CompilerParams(use_tc_tiling_on_sc=False),
  )
  def body(x_hbm, y_hbm, table_hbm, w_hbm, sumx_hbm, wy_hbm,
           idx_v, rows_v, yidx_v, wrows_v, acc_v, sem):
    wid = lax.axis_index("s") * _NC + lax.axis_index("c")
    xbase = wid * idx_per_w
    rbase = wid * rows_per_w

    pltpu.sync_copy(x_hbm.at[pl.ds(xbase, idx_per_w)], idx_v)
    pltpu.sync_copy(y_hbm.at[pl.ds(rbase, rows_per_w)], yidx_v)

    copies = []
    for c in range(n_chunks):
      copies.append(pltpu.async_copy(
          table_hbm.at[idx_v.at[pl.ds(c * chunk, chunk)]],
          rows_v.at[pl.ds(c * chunk, chunk)], sem))
    copies.append(pltpu.async_copy(w_hbm.at[yidx_v], wrows_v, sem))
    for cp in copies:
      cp.wait()

    def accum_row(r, _):
      for c in range(E // 16):
        sl = pl.ds(c * 16, 16)
        a = rows_v[r * CTX, sl]
        for j in range(1, CTX):
          a = a + rows_v[r * CTX + j, sl]
        acc_v[r, sl] = a
      return _

    lax.fori_loop(0, rows_per_w, accum_row, 0)

    pltpu.sync_copy(acc_v, sumx_hbm.at[pl.ds(rbase, rows_per_w)])
    pltpu.sync_copy(wrows_v, wy_hbm.at[pl.ds(rbase, rows_per_w)])

  return body


def _fused_loss(B, V, E, interpret=False):
  assert V % _VB == 0
  nv = V // _VB
  bound = 1.0 / (E ** 0.5)

  ln2 = 0.6931471805599453
  log2e = 1.4426950408889634

  def body(sumx_ref, w_ref, wy_ref, loss_ref, rhs_ref, s_ref):
    v = pl.program_id(0)

    @pl.when(v == 0)
    def _init():
      sx = sumx_ref[...]
      r = lax.broadcasted_iota(jnp.int32, (E, E), 0)
      c = lax.broadcasted_iota(jnp.int32, (E, E), 1)
      eye = (r == c).astype(jnp.float32) * log2e
      sxT = lax.dot_general(eye, sx, (((1,), (1,)), ((), ())),
                            preferred_element_type=jnp.float32)
      ones_row = jnp.ones((1, E), jnp.float32)
      m = (bound * log2e) * lax.dot_general(ones_row, jnp.abs(sx),
                                            (((1,), (1,)), ((), ())),
                                            preferred_element_type=jnp.float32)
      rhs_ref[pl.ds(0, E), :] = sxT.astype(jnp.bfloat16)
      rhs_ref[pl.ds(E, 1), :] = -m.astype(jnp.bfloat16)
      s_ref[...] = jnp.zeros(s_ref.shape, jnp.float32)

    w_aug = jnp.concatenate(
        [w_ref[...].astype(jnp.bfloat16),
         jnp.ones((_VB, 1), jnp.bfloat16)], axis=1)
    zm = lax.dot_general(w_aug, rhs_ref[...], (((1,), (0,)), ((), ())),
                         preferred_element_type=jnp.float32)
    s_ref[...] = s_ref[...] + jnp.sum(jnp.exp2(zm), axis=0, keepdims=True)

    @pl.when(v == nv - 1)
    def _fin():
      ones_row = jnp.ones((1, E), jnp.float32)
      pk = lax.dot_general(ones_row, sumx_ref[...] * wy_ref[...],
                           (((1,), (1,)), ((), ())),
                           preferred_element_type=jnp.float32)
      m = -rhs_ref[pl.ds(E, 1), :].astype(jnp.float32) * ln2
      lse = m + jnp.log(s_ref[...])
      loss_ref[0, 0] = jnp.mean(lse - pk)

  return pl.pallas_call(
      body,
      grid=(nv,),
      in_specs=[
          pl.BlockSpec((B, E), lambda v: (0, 0)),
          pl.BlockSpec((_VB, E), lambda v: (v, 0)),
          pl.BlockSpec((B, E), lambda v: (0, 0)),
      ],
      out_specs=pl.BlockSpec((1, 1), lambda v: (0, 0),
                             memory_space=pltpu.SMEM),
      out_shape=jax.ShapeDtypeStruct((1, 1), jnp.float32),
      scratch_shapes=[pltpu.VMEM((E + 1, B), jnp.bfloat16),
                      pltpu.VMEM((1, B), jnp.float32)],
      interpret=interpret,
  )


def kernel(batch_X, batch_Y, emb_table, linear_W):
  B, CTX = batch_X.shape
  V, E = emb_table.shape
  x_flat = batch_X.reshape(-1).astype(jnp.int32)
  y = batch_Y.astype(jnp.int32)
  sum_x, w_y = _embed_bag(B, CTX, V, E)(x_flat, y, emb_table, linear_W)
  loss = _fused_loss(B, V, E)(sum_x, linear_W, w_y)
  return loss[0, 0]


# speedup vs baseline: 2.5148x; 1.0251x over previous
import functools

import jax
import jax.numpy as jnp
from jax import lax
from jax.experimental import pallas as pl
from jax.experimental.pallas import tpu as pltpu
from jax.experimental.pallas import tpu_sc as plsc

_NC = 2
_NS = 16
_NW = _NC * _NS
_L = 16

_VB = 2000


def _embed_bag(B, CTX, V, E):
  rows_per_w = B // _NW
  idx_per_w = rows_per_w * CTX
  chunk = 64
  n_chunks = idx_per_w // chunk

  mesh = plsc.VectorSubcoreMesh(core_axis_name="c", subcore_axis_name="s")

  @functools.partial(
      pl.kernel,
      out_type=[
          jax.ShapeDtypeStruct((B, E), jnp.float32),
          jax.ShapeDtypeStruct((B, E), jnp.float32),
      ],
      mesh=mesh,
      scratch_types=[
          pltpu.VMEM((idx_per_w,), jnp.int32),
          pltpu.VMEM((idx_per_w, E), jnp.float32),
          pltpu.VMEM((rows_per_w,), jnp.int32),
          pltpu.VMEM((rows_per_w,), jnp.int32),
          pltpu.VMEM((rows_per_w,), jnp.int32),
          pltpu.VMEM((rows_per_w, E // 2), jnp.float32),
          pltpu.VMEM((rows_per_w, E // 2), jnp.float32),
          pltpu.VMEM((rows_per_w, E), jnp.float32),
          pltpu.VMEM((rows_per_w, E), jnp.float32),
          pltpu.SemaphoreType.DMA,
      ],
      compiler_params=pltpu.CompilerParams(use_tc_tiling_on_sc=False),
  )
  def body(x_hbm, y_hbm, table_hbm, wh_hbm, sumx_hbm, wy_hbm,
           idx_v, rows_v, yidx_v, y2_v, y2p1_v, wlo_v, whi_v, wrows_v,
           acc_v, sem):
    wid = lax.axis_index("s") * _NC + lax.axis_index("c")
    xbase = wid * idx_per_w
    rbase = wid * rows_per_w

    pltpu.sync_copy(x_hbm.at[pl.ds(xbase, idx_per_w)], idx_v)
    pltpu.sync_copy(y_hbm.at[pl.ds(rbase, rows_per_w)], yidx_v)

    for c in range(rows_per_w // _L):
      sl = pl.ds(c * _L, _L)
      y2 = lax.shift_left(yidx_v[sl], 1)
      y2_v[sl] = y2
      y2p1_v[sl] = y2 + 1

    copies = []
    for c in range(n_chunks):
      copies.append(pltpu.async_copy(
          table_hbm.at[idx_v.at[pl.ds(c * chunk, chunk)]],
          rows_v.at[pl.ds(c * chunk, chunk)], sem))
    copies.append(pltpu.async_copy(wh_hbm.at[y2_v], wlo_v, sem))
    copies.append(pltpu.async_copy(wh_hbm.at[y2p1_v], whi_v, sem))
    for cp in copies:
      cp.wait()

    def accum_row(r, _):
      for c in range(E // _L):
        sl = pl.ds(c * _L, _L)
        a = rows_v[r * CTX, sl]
        for j in range(1, CTX):
          a = a + rows_v[r * CTX + j, sl]
        acc_v[r, sl] = a
      return _

    lax.fori_loop(0, rows_per_w, accum_row, 0)

    def join_halves(r, _):
      for c in range(E // (2 * _L)):
        sl = pl.ds(c * _L, _L)
        wrows_v[r, pl.ds(c * _L, _L)] = wlo_v[r, sl]
        wrows_v[r, pl.ds(E // 2 + c * _L, _L)] = whi_v[r, sl]
      return _

    lax.fori_loop(0, rows_per_w, join_halves, 0)

    pltpu.sync_copy(acc_v, sumx_hbm.at[pl.ds(rbase, rows_per_w)])
    pltpu.sync_copy(wrows_v, wy_hbm.at[pl.ds(rbase, rows_per_w)])

  return body


def _fused_loss(B, V, E, interpret=False):
  assert V % _VB == 0 and _VB % 2 == 0
  nv = V // _VB
  hb = _VB // 2
  bound = 1.0 / (E ** 0.5)

  ln2 = 0.6931471805599453
  log2e = 1.4426950408889634

  def body(sumx_ref, w_ref, wy_ref, loss_ref, rhs_ref, s_ref):
    v = pl.program_id(0)

    @pl.when(v == 0)
    def _init():
      sx = sumx_ref[...]
      r = lax.broadcasted_iota(jnp.int32, (E, E), 0)
      c = lax.broadcasted_iota(jnp.int32, (E, E), 1)
      eye = (r == c).astype(jnp.float32) * log2e
      sxT = lax.dot_general(eye, sx, (((1,), (1,)), ((), ())),
                            preferred_element_type=jnp.float32)
      ones_row = jnp.ones((1, E), jnp.float32)
      m = (bound * log2e) * lax.dot_general(ones_row, jnp.abs(sx),
                                            (((1,), (1,)), ((), ())),
                                            preferred_element_type=jnp.float32)
      rhs_ref[pl.ds(0, E), :] = sxT.astype(jnp.bfloat16)
      rhs_ref[pl.ds(E, 1), :] = -m.astype(jnp.bfloat16)
      s_ref[...] = jnp.zeros(s_ref.shape, jnp.float32)

    ones_col = jnp.ones((hb, 1), jnp.bfloat16)
    wf = w_ref[...].astype(jnp.bfloat16)
    acc = s_ref[...]
    for half in (wf[:, :E], wf[:, E:]):
      w_aug = jnp.concatenate([half, ones_col], axis=1)
      zm = lax.dot_general(w_aug, rhs_ref[...], (((1,), (0,)), ((), ())),
                           preferred_element_type=jnp.float32)
      acc = acc + jnp.sum(jnp.exp2(zm), axis=0, keepdims=True)
    s_ref[...] = acc

    @pl.when(v == nv - 1)
    def _fin():
      ones_row = jnp.ones((1, E), jnp.float32)
      pk = lax.dot_general(ones_row, sumx_ref[...] * wy_ref[...],
                           (((1,), (1,)), ((), ())),
                           preferred_element_type=jnp.float32)
      m = -rhs_ref[pl.ds(E, 1), :].astype(jnp.float32) * ln2
      lse = m + jnp.log(s_ref[...])
      loss_ref[0, 0] = jnp.mean(lse - pk)

  return pl.pallas_call(
      body,
      grid=(nv,),
      in_specs=[
          pl.BlockSpec((B, E), lambda v: (0, 0)),
          pl.BlockSpec((hb, 2 * E), lambda v: (v, 0)),
          pl.BlockSpec((B, E), lambda v: (0, 0)),
      ],
      out_specs=pl.BlockSpec((1, 1), lambda v: (0, 0),
                             memory_space=pltpu.SMEM),
      out_shape=jax.ShapeDtypeStruct((1, 1), jnp.float32),
      scratch_shapes=[pltpu.VMEM((E + 1, B), jnp.bfloat16),
                      pltpu.VMEM((1, B), jnp.float32)],
      interpret=interpret,
  )


def kernel(batch_X, batch_Y, emb_table, linear_W):
  B, CTX = batch_X.shape
  V, E = emb_table.shape
  x_flat = batch_X.reshape(-1).astype(jnp.int32)
  y = batch_Y.astype(jnp.int32)
  w2 = linear_W.reshape(V // 2, 2 * E)
  w_half = linear_W.reshape(2 * V, E // 2)
  sum_x, w_y = _embed_bag(B, CTX, V, E)(x_flat, y, emb_table, w_half)
  loss = _fused_loss(B, V, E)(sum_x, w2, w_y)
  return loss[0, 0]
